# baseline (speedup 1.0000x reference)
import functools

import jax
import jax.numpy as jnp
from jax import lax
from jax.experimental import pallas as pl
from jax.experimental.pallas import tpu as pltpu
from jax.experimental.pallas import tpu_sc as plsc

NUM_ENTITIES = 1000000
NUM_RELATIONS = 1000
D = 64
BATCH = 16384

NC, NS, L = 2, 16, 16
NW = NC * NS
CHUNK = 128
N_CHUNKS = BATCH // CHUNK
CPW = N_CHUNKS // NW


def _trig_body(rel_ref, cs_ref):
    th = rel_ref[...]
    cs_ref[...] = jnp.concatenate([jnp.cos(th), jnp.sin(th)], axis=1)


_trig = pl.pallas_call(
    _trig_body,
    out_shape=jax.ShapeDtypeStruct((NUM_RELATIONS, 2 * D), jnp.float32),
)


def _rotate_body(sup_ref, r_ref, col_ref, ent_re, ent_im, cs_t,
                 out, idx_e, idx_r, col_v, er, ei, cs, sem):
    wid = lax.axis_index("s") * NC + lax.axis_index("c")
    row0 = wid * CPW
    pltpu.sync_copy(sup_ref.at[pl.ds(row0, CPW)], idx_e)
    pltpu.sync_copy(r_ref.at[pl.ds(row0, CPW)], idx_r)
    pltpu.sync_copy(col_ref.at[pl.ds(row0, CPW)], col_v)
    for j in range(CPW):
        cps = [
            pltpu.async_copy(ent_re.at[idx_e.at[j]], er, sem),
            pltpu.async_copy(ent_im.at[idx_e.at[j]], ei, sem),
            pltpu.async_copy(cs_t.at[idx_r.at[j]], cs, sem),
        ]
        for c in cps:
            c.wait()

        def body(i, carry):
            zl = jnp.zeros((L,), jnp.int32)
            p = plsc.load_gather(col_v, [zl + j, zl + i])
            m = p > 0
            for k in range(D // L):
                sl = pl.ds(k * L, L)
                sh = pl.ds(D + k * L, L)
                a = jnp.where(m, er[i, sh], er[i, sl])
                b = jnp.where(m, ei[i, sh], ei[i, sl])
                c = cs[i, sl]
                s = cs[i, sh]
                cs[i, sl] = a * c - b * s
                cs[i, sh] = a * s + b * c
            return carry

        lax.fori_loop(0, CHUNK, body, 0)
        base = (row0 + j) * CHUNK
        pltpu.sync_copy(cs, out.at[pl.ds(base, CHUNK)])


_rotate = functools.partial(
    pl.kernel,
    out_type=jax.ShapeDtypeStruct((BATCH, 2 * D), jnp.float32),
    mesh=plsc.VectorSubcoreMesh(
        core_axis_name="c", subcore_axis_name="s", num_cores=NC, num_subcores=NS),
    scratch_types=[
        pltpu.VMEM((CPW, CHUNK), jnp.int32),
        pltpu.VMEM((CPW, CHUNK), jnp.int32),
        pltpu.VMEM((CPW, CHUNK), jnp.int32),
        pltpu.VMEM((CHUNK, 2 * D), jnp.float32),
        pltpu.VMEM((CHUNK, 2 * D), jnp.float32),
        pltpu.VMEM((CHUNK, 2 * D), jnp.float32),
        pltpu.SemaphoreType.DMA,
    ],
    compiler_params=pltpu.CompilerParams(needs_layout_passes=False),
)(_rotate_body)


def kernel(e1, r, entity_real, entity_img, relation):
    e1 = e1.astype(jnp.int32)
    r = r.astype(jnp.int32).reshape(N_CHUNKS, CHUNK)
    sup = (e1 >> 1).reshape(N_CHUNKS, CHUNK)
    col = ((e1 & 1) * D).reshape(N_CHUNKS, CHUNK)
    er2 = entity_real.reshape(NUM_ENTITIES // 2, 2 * D)
    ei2 = entity_img.reshape(NUM_ENTITIES // 2, 2 * D)
    cs_t = _trig(relation)
    out = _rotate(sup, r, col, er2, ei2, cs_t)
    return out[:, :D], out[:, D:]
